# Initial kernel scaffold; baseline (speedup 1.0000x reference)
import jax
import jax.numpy as jnp
from jax.experimental import pallas as pl


def kernel(x, w1_t, b1, w2_t, b2):
    raise NotImplementedError("write your pallas kernel here")


# speedup vs baseline: 1.7692x; 1.7692x over previous
import functools

import jax
import jax.numpy as jnp
from jax.experimental import pallas as pl
from jax.experimental.pallas import tpu as pltpu


def _mlp_body(x_ref, w1_ref, b1_ref, w2_ref, b2_ref, o_ref):
    x = x_ref[...]
    h = jnp.maximum(
        jnp.dot(x, w1_ref[...], preferred_element_type=jnp.float32) + b1_ref[...],
        0.0,
    )
    o = jnp.dot(h, w2_ref[...], preferred_element_type=jnp.float32) + b2_ref[...]
    o_ref[...] = o.astype(o_ref.dtype)


@functools.partial(jax.jit, static_argnames=("block_b",))
def _mlp_forward(x, w1_t, b1, w2_t, b2, *, block_b=2048):
    B, f_in = x.shape
    f_out = w2_t.shape[1]

    x = x.astype(jnp.float32)
    w1 = w1_t.astype(jnp.float32)
    w2 = w2_t.astype(jnp.float32)
    b1r = b1.astype(jnp.float32).reshape(1, -1)
    b2r = b2.astype(jnp.float32).reshape(1, -1)

    tb = min(block_b, B)
    b_pad = (B + tb - 1) // tb * tb
    if b_pad != B:
        x = jnp.pad(x, ((0, b_pad - B), (0, 0)))
    nb = b_pad // tb

    out = pl.pallas_call(
        _mlp_body,
        out_shape=jax.ShapeDtypeStruct((b_pad, f_out), jnp.float32),
        grid_spec=pl.GridSpec(
            grid=(nb,),
            in_specs=[
                pl.BlockSpec((tb, f_in), lambda i: (i, 0)),
                pl.BlockSpec((f_in, f_in), lambda i: (0, 0)),
                pl.BlockSpec((1, f_in), lambda i: (0, 0)),
                pl.BlockSpec((f_in, f_out), lambda i: (0, 0)),
                pl.BlockSpec((1, f_out), lambda i: (0, 0)),
            ],
            out_specs=pl.BlockSpec((tb, f_out), lambda i: (i, 0)),
        ),
        compiler_params=pltpu.CompilerParams(
            dimension_semantics=("parallel",),
        ),
    )(x, w1, b1r, w2, b2r)

    if b_pad != B:
        out = out[:B]
    return out


def kernel(x, w1_t, b1, w2_t, b2):
    return _mlp_forward(x, w1_t, b1, w2_t, b2, block_b=2048)
